# baseline (speedup 1.0000x reference)
import functools

import jax
import jax.numpy as jnp
from jax import lax
from jax.experimental import pallas as pl
from jax.experimental.pallas import tpu as pltpu
from jax.experimental.pallas import tpu_sc as plsc

_N = 10000
_E = 320000
_D = 128

_NC = 2
_NS = 16
_NW = _NC * _NS
_EPT = _E // _NW
_RPT = _N // _NS
_TRASH = 8
_K = 2

_CFG = {144: 32, 128: 32}


def _nch(W):
    return -(-_EPT // _CFG[W])


def _make_agg(W, add=True):
    C = _CFG[W]
    NCH = _nch(W)
    mesh = plsc.VectorSubcoreMesh(core_axis_name="c", subcore_axis_name="s")

    @functools.partial(
        pl.kernel,
        mesh=mesh,
        out_type=jax.ShapeDtypeStruct((_NC, _N, W), jnp.float32),
        scratch_types=[
            pltpu.VMEM((NCH, C), jnp.int32),
            pltpu.VMEM((NCH, C), jnp.int32),
            pltpu.VMEM((_K, C, W), jnp.float32),
            pltpu.SemaphoreType.DMA((_K,)),
            pltpu.VMEM_SHARED((_N + _TRASH, W), jnp.float32),
        ],
        compiler_params=pltpu.CompilerParams(use_tc_tiling_on_sc=False),
    )
    def agg(feat_hbm, src_hbm, dst_hbm, zeros_hbm, out_hbm, src_v, dst_v,
            rows_v, sems, acc):
        cid = lax.axis_index("c")
        sid = lax.axis_index("s")
        wid = cid * _NS + sid

        pltpu.sync_copy(zeros_hbm.at[pl.ds(0, _RPT)], acc.at[pl.ds(sid * _RPT, _RPT)])

        @pl.when(sid == 0)
        def _():
            pltpu.sync_copy(zeros_hbm.at[pl.ds(0, _TRASH)], acc.at[pl.ds(_N, _TRASH)])

        pltpu.sync_copy(src_hbm.at[wid], src_v)
        pltpu.sync_copy(dst_hbm.at[wid], dst_v)
        plsc.subcore_barrier()

        for j in range(_K - 1):
            pltpu.async_copy(feat_hbm.at[src_v.at[j]], rows_v.at[j], sems.at[j])

        @pl.loop(0, NCH)
        def body(j):
            p = lax.rem(j, _K)
            q = lax.rem(j + _K - 1, _K)

            @pl.when(j + _K - 1 < NCH)
            def _():
                pltpu.async_copy(feat_hbm.at[src_v.at[j + _K - 1]],
                                 rows_v.at[q], sems.at[q])

            pltpu.make_async_copy(feat_hbm.at[src_v.at[j]], rows_v.at[p],
                                  sems.at[p]).wait()
            pltpu.sync_copy(rows_v.at[p], acc.at[dst_v.at[j]], add=add)

        plsc.subcore_barrier()

        pltpu.sync_copy(acc.at[pl.ds(sid * _RPT, _RPT)],
                        out_hbm.at[cid, pl.ds(sid * _RPT, _RPT)])

    return agg


_R = 400


def _dense_body(h_ref, p_ref, inv_ref, ws_ref, wn_ref, b_ref, *rest, relu):
    if relu:
        g_ref, be_ref, out_ref = rest
    else:
        (out_ref,) = rest
    p = p_ref[0] + p_ref[1]
    neigh = p[:, :_D] * inv_ref[...]
    h = h_ref[...]
    out = (jnp.dot(h, ws_ref[...], preferred_element_type=jnp.float32)
           + jnp.dot(neigh, wn_ref[...], preferred_element_type=jnp.float32)
           + b_ref[...] + h)
    if relu:
        out = jnp.maximum(g_ref[...] * out + be_ref[...], 0.0)
    out_ref[...] = out


def _dense0_body(h_ref, p_ref, ws_ref, wn_ref, b_ref, g_ref, be_ref,
                 out_ref, inv_ref):
    p = p_ref[0] + p_ref[1]
    inv = 1.0 / jnp.maximum(p[:, _D:_D + 1], 1.0)
    neigh = p[:, :_D] * inv
    h = h_ref[...]
    out = (jnp.dot(h, ws_ref[...], preferred_element_type=jnp.float32)
           + jnp.dot(neigh, wn_ref[...], preferred_element_type=jnp.float32)
           + b_ref[...] + h)
    out_ref[...] = jnp.maximum(g_ref[...] * out + be_ref[...], 0.0)
    inv_ref[...] = jnp.broadcast_to(inv, (_R, _D))


def _full(shape):
    return pl.BlockSpec(shape, lambda i: (0,) * len(shape))


def _rows():
    return pl.BlockSpec((_R, _D), lambda i: (i, 0))


def _dense0(h, p, ws, wn, b, g, be):
    return pl.pallas_call(
        _dense0_body,
        grid=(_N // _R,),
        in_specs=[
            _rows(),
            pl.BlockSpec((_NC, _R, _D + 16), lambda i: (0, i, 0)),
            _full((_D, _D)), _full((_D, _D)),
            _full((1, _D)), _full((1, _D)), _full((1, _D)),
        ],
        out_specs=[_rows(), _rows()],
        out_shape=[jax.ShapeDtypeStruct((_N, _D), jnp.float32),
                   jax.ShapeDtypeStruct((_N, _D), jnp.float32)],
    )(h, p, ws, wn, b, g, be)


def _dense(h, p, inv, ws, wn, b, g, be, relu):
    body = functools.partial(_dense_body, relu=relu)
    n_aff = 2 if relu else 0
    affine = (g, be) if relu else ()
    return pl.pallas_call(
        body,
        grid=(_N // _R,),
        in_specs=[
            _rows(),
            pl.BlockSpec((_NC, _R, _D), lambda i: (0, i, 0)),
            _rows(),
            _full((_D, _D)), _full((_D, _D)),
            _full((1, _D)),
        ] + [_full((1, _D))] * n_aff,
        out_specs=_rows(),
        out_shape=jax.ShapeDtypeStruct((_N, _D), jnp.float32),
    )(h, p, inv, ws, wn, b, *affine)


_agg144 = _make_agg(_D + 16)
_agg128 = _make_agg(_D)


def kernel(feat, edge_index, W_self_0, W_neigh_0, b_0, W_self_1, W_neigh_1,
           b_1, W_self_2, W_neigh_2, b_2, gamma_0, beta_0, gamma_1, beta_1):
    src = edge_index[0]
    dst = edge_index[1]

    def layout(W):
        C = _CFG[W]
        NCH = _nch(W)
        pad = NCH * C
        s3 = (jnp.zeros((_NW, pad), jnp.int32)
              .at[:, :_EPT].set(src.reshape(_NW, _EPT))
              .reshape(_NW, NCH, C))
        d3 = (jnp.full((_NW, pad), _N, jnp.int32)
              .at[:, :_EPT].set(dst.reshape(_NW, _EPT))
              .reshape(_NW, NCH, C))
        return s3, d3

    src_a, dst_a = layout(_D + 16)
    src_b, dst_b = src_a, dst_a
    z144 = jnp.zeros((_RPT, _D + 16), jnp.float32)
    z128 = jnp.zeros((_RPT, _D), jnp.float32)

    feat_aug = jnp.concatenate(
        [feat, jnp.ones((_N, 1), jnp.float32), jnp.zeros((_N, 15), jnp.float32)],
        axis=1)

    b0 = b_0.reshape(1, _D)
    b1 = b_1.reshape(1, _D)
    b2 = b_2.reshape(1, _D)
    g0 = gamma_0.reshape(1, _D)
    be0 = beta_0.reshape(1, _D)
    g1 = gamma_1.reshape(1, _D)
    be1 = beta_1.reshape(1, _D)

    p0 = _agg144(feat_aug, src_a, dst_a, z144)
    h1, inv = _dense0(feat, p0, W_self_0, W_neigh_0, b0, g0, be0)
    p1 = _agg128(h1, src_b, dst_b, z128)
    h2 = _dense(h1, p1, inv, W_self_1, W_neigh_1, b1, g1, be1, relu=True)
    p2 = _agg128(h2, src_b, dst_b, z128)
    h3 = _dense(h2, p2, inv, W_self_2, W_neigh_2, b2, None, None, relu=False)
    return h3


# speedup vs baseline: 1.1173x; 1.1173x over previous
import functools

import jax
import jax.numpy as jnp
from jax import lax
from jax.experimental import pallas as pl
from jax.experimental.pallas import tpu as pltpu
from jax.experimental.pallas import tpu_sc as plsc

_N = 10000
_E = 320000
_D = 128

_NC = 2
_NS = 16
_NW = _NC * _NS
_EPT = _E // _NW
_RPT = _N // _NS
_TRASH = 8
_K = 2

_CFG = {144: 64, 128: 64}


def _nch(W):
    return -(-_EPT // _CFG[W])


def _make_agg(W, K=_K, add=True):
    C = _CFG[W]
    NCH = _nch(W)
    mesh = plsc.VectorSubcoreMesh(core_axis_name="c", subcore_axis_name="s")

    @functools.partial(
        pl.kernel,
        mesh=mesh,
        out_type=jax.ShapeDtypeStruct((_NC, _N, W), jnp.float32),
        scratch_types=[
            pltpu.VMEM((NCH, C), jnp.int32),
            pltpu.VMEM((NCH, C), jnp.int32),
            pltpu.VMEM((K, C, W), jnp.float32),
            pltpu.SemaphoreType.DMA((K,)),
            pltpu.VMEM_SHARED((_N + _TRASH, W), jnp.float32),
        ],
        compiler_params=pltpu.CompilerParams(use_tc_tiling_on_sc=False),
    )
    def agg(feat_hbm, src_hbm, dst_hbm, zeros_hbm, out_hbm, src_v, dst_v,
            rows_v, sems, acc):
        cid = lax.axis_index("c")
        sid = lax.axis_index("s")
        wid = cid * _NS + sid

        pltpu.sync_copy(zeros_hbm.at[pl.ds(0, _RPT)], acc.at[pl.ds(sid * _RPT, _RPT)])

        @pl.when(sid == 0)
        def _():
            pltpu.sync_copy(zeros_hbm.at[pl.ds(0, _TRASH)], acc.at[pl.ds(_N, _TRASH)])

        pltpu.sync_copy(src_hbm.at[wid], src_v)
        pltpu.sync_copy(dst_hbm.at[wid], dst_v)
        plsc.subcore_barrier()

        for j in range(K - 1):
            pltpu.async_copy(feat_hbm.at[src_v.at[j]], rows_v.at[j], sems.at[j])

        @pl.loop(0, NCH)
        def body(j):
            p = lax.rem(j, K)
            q = lax.rem(j + K - 1, K)

            @pl.when(j + K - 1 < NCH)
            def _():
                pltpu.async_copy(feat_hbm.at[src_v.at[j + K - 1]],
                                 rows_v.at[q], sems.at[q])

            pltpu.make_async_copy(feat_hbm.at[src_v.at[j]], rows_v.at[p],
                                  sems.at[p]).wait()
            pltpu.sync_copy(rows_v.at[p], acc.at[dst_v.at[j]], add=add)

        plsc.subcore_barrier()

        pltpu.sync_copy(acc.at[pl.ds(sid * _RPT, _RPT)],
                        out_hbm.at[cid, pl.ds(sid * _RPT, _RPT)])

    return agg


_R = 400


def _dense_body(h_ref, p_ref, inv_ref, ws_ref, wn_ref, b_ref, *rest, relu):
    if relu:
        g_ref, be_ref, out_ref = rest
    else:
        (out_ref,) = rest
    p = p_ref[0] + p_ref[1]
    neigh = p[:, :_D] * inv_ref[...]
    h = h_ref[...]
    out = (jnp.dot(h, ws_ref[...], preferred_element_type=jnp.float32)
           + jnp.dot(neigh, wn_ref[...], preferred_element_type=jnp.float32)
           + b_ref[...] + h)
    if relu:
        out = jnp.maximum(g_ref[...] * out + be_ref[...], 0.0)
    out_ref[...] = out


def _dense0_body(h_ref, p_ref, ws_ref, wn_ref, b_ref, g_ref, be_ref,
                 out_ref, inv_ref):
    p = p_ref[0] + p_ref[1]
    inv = 1.0 / jnp.maximum(p[:, _D:_D + 1], 1.0)
    neigh = p[:, :_D] * inv
    h = h_ref[...]
    out = (jnp.dot(h, ws_ref[...], preferred_element_type=jnp.float32)
           + jnp.dot(neigh, wn_ref[...], preferred_element_type=jnp.float32)
           + b_ref[...] + h)
    out_ref[...] = jnp.maximum(g_ref[...] * out + be_ref[...], 0.0)
    inv_ref[...] = jnp.broadcast_to(inv, (_R, _D))


def _full(shape):
    return pl.BlockSpec(shape, lambda i: (0,) * len(shape))


def _rows():
    return pl.BlockSpec((_R, _D), lambda i: (i, 0))


def _dense0(h, p, ws, wn, b, g, be):
    return pl.pallas_call(
        _dense0_body,
        grid=(_N // _R,),
        in_specs=[
            _rows(),
            pl.BlockSpec((_NC, _R, _D + 16), lambda i: (0, i, 0)),
            _full((_D, _D)), _full((_D, _D)),
            _full((1, _D)), _full((1, _D)), _full((1, _D)),
        ],
        out_specs=[_rows(), _rows()],
        out_shape=[jax.ShapeDtypeStruct((_N, _D), jnp.float32),
                   jax.ShapeDtypeStruct((_N, _D), jnp.float32)],
    )(h, p, ws, wn, b, g, be)


def _dense(h, p, inv, ws, wn, b, g, be, relu):
    body = functools.partial(_dense_body, relu=relu)
    n_aff = 2 if relu else 0
    affine = (g, be) if relu else ()
    return pl.pallas_call(
        body,
        grid=(_N // _R,),
        in_specs=[
            _rows(),
            pl.BlockSpec((_NC, _R, _D), lambda i: (0, i, 0)),
            _rows(),
            _full((_D, _D)), _full((_D, _D)),
            _full((1, _D)),
        ] + [_full((1, _D))] * n_aff,
        out_specs=_rows(),
        out_shape=jax.ShapeDtypeStruct((_N, _D), jnp.float32),
    )(h, p, inv, ws, wn, b, *affine)


_agg144 = _make_agg(_D + 16)
_agg128 = _make_agg(_D, K=3)


def kernel(feat, edge_index, W_self_0, W_neigh_0, b_0, W_self_1, W_neigh_1,
           b_1, W_self_2, W_neigh_2, b_2, gamma_0, beta_0, gamma_1, beta_1):
    src = edge_index[0]
    dst = edge_index[1]

    def layout(W):
        C = _CFG[W]
        NCH = _nch(W)
        pad = NCH * C
        s3 = (jnp.zeros((_NW, pad), jnp.int32)
              .at[:, :_EPT].set(src.reshape(_NW, _EPT))
              .reshape(_NW, NCH, C))
        d3 = (jnp.full((_NW, pad), _N, jnp.int32)
              .at[:, :_EPT].set(dst.reshape(_NW, _EPT))
              .reshape(_NW, NCH, C))
        return s3, d3

    src_a, dst_a = layout(_D + 16)
    src_b, dst_b = src_a, dst_a
    z144 = jnp.zeros((_RPT, _D + 16), jnp.float32)
    z128 = jnp.zeros((_RPT, _D), jnp.float32)

    feat_aug = jnp.concatenate(
        [feat, jnp.ones((_N, 1), jnp.float32), jnp.zeros((_N, 15), jnp.float32)],
        axis=1)

    b0 = b_0.reshape(1, _D)
    b1 = b_1.reshape(1, _D)
    b2 = b_2.reshape(1, _D)
    g0 = gamma_0.reshape(1, _D)
    be0 = beta_0.reshape(1, _D)
    g1 = gamma_1.reshape(1, _D)
    be1 = beta_1.reshape(1, _D)

    p0 = _agg144(feat_aug, src_a, dst_a, z144)
    h1, inv = _dense0(feat, p0, W_self_0, W_neigh_0, b0, g0, be0)
    p1 = _agg128(h1, src_b, dst_b, z128)
    h2 = _dense(h1, p1, inv, W_self_1, W_neigh_1, b1, g1, be1, relu=True)
    p2 = _agg128(h2, src_b, dst_b, z128)
    h3 = _dense(h2, p2, inv, W_self_2, W_neigh_2, b2, None, None, relu=False)
    return h3


# speedup vs baseline: 1.1845x; 1.0602x over previous
import functools

import jax
import jax.numpy as jnp
from jax import lax
from jax.experimental import pallas as pl
from jax.experimental.pallas import tpu as pltpu
from jax.experimental.pallas import tpu_sc as plsc

_N = 10000
_E = 320000
_D = 128

_NC = 2
_NS = 16
_NW = _NC * _NS
_EPT = _E // _NW
_RPT = _N // _NS
_TRASH = 8
_K = 2

_CFG = {144: 48, 128: 64}


def _nch(W):
    return -(-_EPT // _CFG[W])


def _make_agg(W, K=_K, add=True):
    C = _CFG[W]
    NCH = _nch(W)
    mesh = plsc.VectorSubcoreMesh(core_axis_name="c", subcore_axis_name="s")

    @functools.partial(
        pl.kernel,
        mesh=mesh,
        out_type=jax.ShapeDtypeStruct((_NC, _N, W), jnp.float32),
        scratch_types=[
            pltpu.VMEM((NCH, C), jnp.int32),
            pltpu.VMEM((NCH, C), jnp.int32),
            pltpu.VMEM((K, C, W), jnp.float32),
            pltpu.SemaphoreType.DMA((K,)),
            pltpu.VMEM_SHARED((_N + _TRASH, W), jnp.float32),
        ],
        compiler_params=pltpu.CompilerParams(use_tc_tiling_on_sc=False),
    )
    def agg(feat_hbm, src_hbm, dst_hbm, zeros_hbm, out_hbm, src_v, dst_v,
            rows_v, sems, acc):
        cid = lax.axis_index("c")
        sid = lax.axis_index("s")
        wid = cid * _NS + sid

        pltpu.sync_copy(zeros_hbm.at[pl.ds(0, _RPT)], acc.at[pl.ds(sid * _RPT, _RPT)])

        @pl.when(sid == 0)
        def _():
            pltpu.sync_copy(zeros_hbm.at[pl.ds(0, _TRASH)], acc.at[pl.ds(_N, _TRASH)])

        pltpu.sync_copy(src_hbm.at[wid], src_v)
        pltpu.sync_copy(dst_hbm.at[wid], dst_v)
        plsc.subcore_barrier()

        for j in range(K - 1):
            pltpu.async_copy(feat_hbm.at[src_v.at[j]], rows_v.at[j], sems.at[j])

        @pl.loop(0, NCH)
        def body(j):
            p = lax.rem(j, K)
            q = lax.rem(j + K - 1, K)

            @pl.when(j + K - 1 < NCH)
            def _():
                pltpu.async_copy(feat_hbm.at[src_v.at[j + K - 1]],
                                 rows_v.at[q], sems.at[q])

            pltpu.make_async_copy(feat_hbm.at[src_v.at[j]], rows_v.at[p],
                                  sems.at[p]).wait()
            pltpu.sync_copy(rows_v.at[p], acc.at[dst_v.at[j]], add=add)

        plsc.subcore_barrier()

        pltpu.sync_copy(acc.at[pl.ds(sid * _RPT, _RPT)],
                        out_hbm.at[cid, pl.ds(sid * _RPT, _RPT)])

    return agg


_R = 400


def _dense_body(h_ref, p_ref, inv_ref, ws_ref, wn_ref, b_ref, *rest, relu):
    if relu:
        g_ref, be_ref, out_ref = rest
    else:
        (out_ref,) = rest
    p = p_ref[0] + p_ref[1]
    neigh = p[:, :_D] * inv_ref[...]
    h = h_ref[...]
    out = (jnp.dot(h, ws_ref[...], preferred_element_type=jnp.float32)
           + jnp.dot(neigh, wn_ref[...], preferred_element_type=jnp.float32)
           + b_ref[...] + h)
    if relu:
        out = jnp.maximum(g_ref[...] * out + be_ref[...], 0.0)
    out_ref[...] = out


def _dense0_body(h_ref, p_ref, ws_ref, wn_ref, b_ref, g_ref, be_ref,
                 out_ref, inv_ref):
    p = p_ref[0] + p_ref[1]
    inv = 1.0 / jnp.maximum(p[:, _D:_D + 1], 1.0)
    neigh = p[:, :_D] * inv
    h = h_ref[...]
    out = (jnp.dot(h, ws_ref[...], preferred_element_type=jnp.float32)
           + jnp.dot(neigh, wn_ref[...], preferred_element_type=jnp.float32)
           + b_ref[...] + h)
    out_ref[...] = jnp.maximum(g_ref[...] * out + be_ref[...], 0.0)
    inv_ref[...] = jnp.broadcast_to(inv, (_R, _D))


def _full(shape):
    return pl.BlockSpec(shape, lambda i: (0,) * len(shape))


def _rows():
    return pl.BlockSpec((_R, _D), lambda i: (i, 0))


def _dense0(h, p, ws, wn, b, g, be):
    return pl.pallas_call(
        _dense0_body,
        grid=(_N // _R,),
        in_specs=[
            _rows(),
            pl.BlockSpec((_NC, _R, _D + 16), lambda i: (0, i, 0)),
            _full((_D, _D)), _full((_D, _D)),
            _full((1, _D)), _full((1, _D)), _full((1, _D)),
        ],
        out_specs=[_rows(), _rows()],
        out_shape=[jax.ShapeDtypeStruct((_N, _D), jnp.float32),
                   jax.ShapeDtypeStruct((_N, _D), jnp.float32)],
    )(h, p, ws, wn, b, g, be)


def _dense(h, p, inv, ws, wn, b, g, be, relu):
    body = functools.partial(_dense_body, relu=relu)
    n_aff = 2 if relu else 0
    affine = (g, be) if relu else ()
    return pl.pallas_call(
        body,
        grid=(_N // _R,),
        in_specs=[
            _rows(),
            pl.BlockSpec((_NC, _R, _D), lambda i: (0, i, 0)),
            _rows(),
            _full((_D, _D)), _full((_D, _D)),
            _full((1, _D)),
        ] + [_full((1, _D))] * n_aff,
        out_specs=_rows(),
        out_shape=jax.ShapeDtypeStruct((_N, _D), jnp.float32),
    )(h, p, inv, ws, wn, b, *affine)


_agg144 = _make_agg(_D + 16, K=3)
_agg128 = _make_agg(_D, K=3)


def kernel(feat, edge_index, W_self_0, W_neigh_0, b_0, W_self_1, W_neigh_1,
           b_1, W_self_2, W_neigh_2, b_2, gamma_0, beta_0, gamma_1, beta_1):
    src = edge_index[0]
    dst = edge_index[1]

    def layout(W):
        C = _CFG[W]
        NCH = _nch(W)
        pad = NCH * C
        s3 = (jnp.zeros((_NW, pad), jnp.int32)
              .at[:, :_EPT].set(src.reshape(_NW, _EPT))
              .reshape(_NW, NCH, C))
        d3 = (jnp.full((_NW, pad), _N, jnp.int32)
              .at[:, :_EPT].set(dst.reshape(_NW, _EPT))
              .reshape(_NW, NCH, C))
        return s3, d3

    src_a, dst_a = layout(_D + 16)
    src_b, dst_b = layout(_D)
    z144 = jnp.zeros((_RPT, _D + 16), jnp.float32)
    z128 = jnp.zeros((_RPT, _D), jnp.float32)

    feat_aug = jnp.concatenate(
        [feat, jnp.ones((_N, 1), jnp.float32), jnp.zeros((_N, 15), jnp.float32)],
        axis=1)

    b0 = b_0.reshape(1, _D)
    b1 = b_1.reshape(1, _D)
    b2 = b_2.reshape(1, _D)
    g0 = gamma_0.reshape(1, _D)
    be0 = beta_0.reshape(1, _D)
    g1 = gamma_1.reshape(1, _D)
    be1 = beta_1.reshape(1, _D)

    p0 = _agg144(feat_aug, src_a, dst_a, z144)
    h1, inv = _dense0(feat, p0, W_self_0, W_neigh_0, b0, g0, be0)
    p1 = _agg128(h1, src_b, dst_b, z128)
    h2 = _dense(h1, p1, inv, W_self_1, W_neigh_1, b1, g1, be1, relu=True)
    p2 = _agg128(h2, src_b, dst_b, z128)
    h3 = _dense(h2, p2, inv, W_self_2, W_neigh_2, b2, None, None, relu=False)
    return h3


# speedup vs baseline: 1.3879x; 1.1717x over previous
import functools

import jax
import jax.numpy as jnp
from jax import lax
from jax.experimental import pallas as pl
from jax.experimental.pallas import tpu as pltpu
from jax.experimental.pallas import tpu_sc as plsc

_N = 10000
_E = 320000
_D = 128

_NC = 2
_NS = 16
_NW = _NC * _NS
_EPT = _E // _NW
_RPT = _N // _NS
_TRASH = 8
_K = 2

_CFG = {144: 32, 128: 56}


def _nch(W):
    return -(-_EPT // _CFG[W])


def _make_agg(W, K=_K, add=True):
    C = _CFG[W]
    NCH = _nch(W)
    mesh = plsc.VectorSubcoreMesh(core_axis_name="c", subcore_axis_name="s")

    @functools.partial(
        pl.kernel,
        mesh=mesh,
        out_type=jax.ShapeDtypeStruct((_NC, _N, W), jnp.float32),
        scratch_types=[
            pltpu.VMEM((NCH, C), jnp.int32),
            pltpu.VMEM((NCH, C), jnp.int32),
            pltpu.VMEM((K, C, W), jnp.float32),
            pltpu.SemaphoreType.DMA((K,)),
            pltpu.VMEM_SHARED((_N + _TRASH, W), jnp.float32),
        ],
        compiler_params=pltpu.CompilerParams(use_tc_tiling_on_sc=False),
    )
    def agg(feat_hbm, src_hbm, dst_hbm, zeros_hbm, out_hbm, src_v, dst_v,
            rows_v, sems, acc):
        cid = lax.axis_index("c")
        sid = lax.axis_index("s")
        wid = cid * _NS + sid

        pltpu.sync_copy(zeros_hbm.at[pl.ds(0, _RPT)], acc.at[pl.ds(sid * _RPT, _RPT)])

        @pl.when(sid == 0)
        def _():
            pltpu.sync_copy(zeros_hbm.at[pl.ds(0, _TRASH)], acc.at[pl.ds(_N, _TRASH)])

        pltpu.sync_copy(src_hbm.at[wid], src_v)
        pltpu.sync_copy(dst_hbm.at[wid], dst_v)
        plsc.subcore_barrier()

        for j in range(K - 1):
            pltpu.async_copy(feat_hbm.at[src_v.at[j]], rows_v.at[j], sems.at[j])

        @pl.loop(0, NCH)
        def body(j):
            p = lax.rem(j, K)
            q = lax.rem(j + K - 1, K)

            @pl.when(j + K - 1 < NCH)
            def _():
                pltpu.async_copy(feat_hbm.at[src_v.at[j + K - 1]],
                                 rows_v.at[q], sems.at[q])

            pltpu.make_async_copy(feat_hbm.at[src_v.at[j]], rows_v.at[p],
                                  sems.at[p]).wait()
            pltpu.sync_copy(rows_v.at[p], acc.at[dst_v.at[j]], add=add)

        plsc.subcore_barrier()

        pltpu.sync_copy(acc.at[pl.ds(sid * _RPT, _RPT)],
                        out_hbm.at[cid, pl.ds(sid * _RPT, _RPT)])

    return agg


_R = 400


def _dense_body(h_ref, p_ref, inv_ref, ws_ref, wn_ref, b_ref, *rest, relu):
    if relu:
        g_ref, be_ref, out_ref = rest
    else:
        (out_ref,) = rest
    p = p_ref[0] + p_ref[1]
    neigh = p[:, :_D] * inv_ref[...]
    h = h_ref[...]
    out = (jnp.dot(h, ws_ref[...], preferred_element_type=jnp.float32)
           + jnp.dot(neigh, wn_ref[...], preferred_element_type=jnp.float32)
           + b_ref[...] + h)
    if relu:
        out = jnp.maximum(g_ref[...] * out + be_ref[...], 0.0)
    out_ref[...] = out


def _dense0_body(h_ref, p_ref, ws_ref, wn_ref, b_ref, g_ref, be_ref,
                 out_ref, inv_ref):
    p = p_ref[0] + p_ref[1]
    inv = 1.0 / jnp.maximum(p[:, _D:_D + 1], 1.0)
    neigh = p[:, :_D] * inv
    h = h_ref[...]
    out = (jnp.dot(h, ws_ref[...], preferred_element_type=jnp.float32)
           + jnp.dot(neigh, wn_ref[...], preferred_element_type=jnp.float32)
           + b_ref[...] + h)
    out_ref[...] = jnp.maximum(g_ref[...] * out + be_ref[...], 0.0)
    inv_ref[...] = jnp.broadcast_to(inv, (_R, _D))


def _full(shape):
    return pl.BlockSpec(shape, lambda i: (0,) * len(shape))


def _rows():
    return pl.BlockSpec((_R, _D), lambda i: (i, 0))


def _dense0(h, p, ws, wn, b, g, be):
    return pl.pallas_call(
        _dense0_body,
        grid=(_N // _R,),
        in_specs=[
            _rows(),
            pl.BlockSpec((_NC, _R, _D + 16), lambda i: (0, i, 0)),
            _full((_D, _D)), _full((_D, _D)),
            _full((1, _D)), _full((1, _D)), _full((1, _D)),
        ],
        out_specs=[_rows(), _rows()],
        out_shape=[jax.ShapeDtypeStruct((_N, _D), jnp.float32),
                   jax.ShapeDtypeStruct((_N, _D), jnp.float32)],
    )(h, p, ws, wn, b, g, be)


def _dense(h, p, inv, ws, wn, b, g, be, relu):
    body = functools.partial(_dense_body, relu=relu)
    n_aff = 2 if relu else 0
    affine = (g, be) if relu else ()
    return pl.pallas_call(
        body,
        grid=(_N // _R,),
        in_specs=[
            _rows(),
            pl.BlockSpec((_NC, _R, _D), lambda i: (0, i, 0)),
            _rows(),
            _full((_D, _D)), _full((_D, _D)),
            _full((1, _D)),
        ] + [_full((1, _D))] * n_aff,
        out_specs=_rows(),
        out_shape=jax.ShapeDtypeStruct((_N, _D), jnp.float32),
    )(h, p, inv, ws, wn, b, *affine)


_agg144 = _make_agg(_D + 16, K=4)
_agg128 = _make_agg(_D, K=4)


def kernel(feat, edge_index, W_self_0, W_neigh_0, b_0, W_self_1, W_neigh_1,
           b_1, W_self_2, W_neigh_2, b_2, gamma_0, beta_0, gamma_1, beta_1):
    src = edge_index[0]
    dst = edge_index[1]

    def layout(W):
        C = _CFG[W]
        NCH = _nch(W)
        pad = NCH * C
        s3 = (jnp.zeros((_NW, pad), jnp.int32)
              .at[:, :_EPT].set(src.reshape(_NW, _EPT))
              .reshape(_NW, NCH, C))
        d3 = (jnp.full((_NW, pad), _N, jnp.int32)
              .at[:, :_EPT].set(dst.reshape(_NW, _EPT))
              .reshape(_NW, NCH, C))
        return s3, d3

    src_a, dst_a = layout(_D + 16)
    src_b, dst_b = layout(_D)
    z144 = jnp.zeros((_RPT, _D + 16), jnp.float32)
    z128 = jnp.zeros((_RPT, _D), jnp.float32)

    feat_aug = jnp.concatenate(
        [feat, jnp.ones((_N, 1), jnp.float32), jnp.zeros((_N, 15), jnp.float32)],
        axis=1)

    b0 = b_0.reshape(1, _D)
    b1 = b_1.reshape(1, _D)
    b2 = b_2.reshape(1, _D)
    g0 = gamma_0.reshape(1, _D)
    be0 = beta_0.reshape(1, _D)
    g1 = gamma_1.reshape(1, _D)
    be1 = beta_1.reshape(1, _D)

    p0 = _agg144(feat_aug, src_a, dst_a, z144)
    h1, inv = _dense0(feat, p0, W_self_0, W_neigh_0, b0, g0, be0)
    p1 = _agg128(h1, src_b, dst_b, z128)
    h2 = _dense(h1, p1, inv, W_self_1, W_neigh_1, b1, g1, be1, relu=True)
    p2 = _agg128(h2, src_b, dst_b, z128)
    h3 = _dense(h2, p2, inv, W_self_2, W_neigh_2, b2, None, None, relu=False)
    return h3
